# baseline (speedup 1.0000x reference)
import jax
import jax.numpy as jnp
from jax import lax
from jax.experimental import pallas as pl
from jax.experimental.pallas import tpu as pltpu
from jax.experimental.pallas import tpu_sc as plsc

VOCAB = 1000000
DIM = 64
IDX_W = 128
GATHERS = 4
CHUNK = IDX_W * GATHERS


def _make_kernel(B, NC, NS):
    NW = NC * NS
    rows_per_w = B // NW
    chunks_per_w = rows_per_w // CHUNK
    idx_rows_per_w = rows_per_w // IDX_W

    mesh = plsc.VectorSubcoreMesh(
        core_axis_name="c", subcore_axis_name="s",
        num_cores=NC, num_subcores=NS)

    @pl.kernel(
        out_type=jax.ShapeDtypeStruct((B, DIM), jnp.float32),
        mesh=mesh,
        scratch_types=[
            pltpu.VMEM((GATHERS, IDX_W), jnp.int32),
            pltpu.VMEM((CHUNK, DIM), jnp.float32),
            pltpu.SemaphoreType.DMA,
        ],
        compiler_params=pltpu.CompilerParams(use_tc_tiling_on_sc=False),
    )
    def k(table_hbm, idx_hbm, out_hbm, idx_v, rows_v, sem):
        wid = lax.axis_index("s") * NC + lax.axis_index("c")
        idx_row0 = wid * idx_rows_per_w
        out_row0 = wid * rows_per_w

        def body(c, carry):
            pltpu.sync_copy(idx_hbm.at[pl.ds(idx_row0 + c * GATHERS, GATHERS)],
                            idx_v)
            for g in range(GATHERS):
                pltpu.async_copy(table_hbm.at[idx_v.at[g]],
                                 rows_v.at[pl.ds(g * IDX_W, IDX_W)], sem)
            for g in range(GATHERS):
                pltpu.make_async_copy(
                    table_hbm.at[idx_v.at[g]],
                    rows_v.at[pl.ds(g * IDX_W, IDX_W)], sem).wait()
            pltpu.sync_copy(rows_v,
                            out_hbm.at[pl.ds(out_row0 + c * CHUNK, CHUNK)])
            return carry

        lax.fori_loop(0, chunks_per_w, body, 0)

    return k


def kernel(input_ids, table):
    BATCH, SEQ = input_ids.shape
    B = BATCH * SEQ
    info = plsc.get_sparse_core_info()
    NC, NS = info.num_cores, info.num_subcores
    idx2d = input_ids.reshape(B // IDX_W, IDX_W)
    k = _make_kernel(B, NC, NS)
    out = k(table, idx2d)
    return out.reshape(BATCH, SEQ, DIM)


# speedup vs baseline: 1.0416x; 1.0416x over previous
import jax
import jax.numpy as jnp
from jax import lax
from jax.experimental import pallas as pl
from jax.experimental.pallas import tpu as pltpu
from jax.experimental.pallas import tpu_sc as plsc

DIM = 64
IDX_W = 128
GATHERS = 4
CHUNK = IDX_W * GATHERS
NBUF = 2


def _make_kernel(B, NC, NS):
    NW = NC * NS
    rows_per_w = B // NW
    chunks_per_w = rows_per_w // CHUNK
    idx_rows_per_w = rows_per_w // IDX_W

    mesh = plsc.VectorSubcoreMesh(
        core_axis_name="c", subcore_axis_name="s",
        num_cores=NC, num_subcores=NS)

    @pl.kernel(
        out_type=jax.ShapeDtypeStruct((B, DIM), jnp.float32),
        mesh=mesh,
        scratch_types=[
            pltpu.VMEM((NBUF, GATHERS, IDX_W), jnp.int32),
            pltpu.VMEM((NBUF, CHUNK, DIM), jnp.float32),
            pltpu.SemaphoreType.DMA((NBUF,)),
            pltpu.SemaphoreType.DMA((NBUF,)),
            pltpu.SemaphoreType.DMA((NBUF,)),
        ],
        compiler_params=pltpu.CompilerParams(use_tc_tiling_on_sc=False),
    )
    def k(table_hbm, idx_hbm, out_hbm, idx_v, rows_v, sem_i, sem_g, sem_w):
        wid = lax.axis_index("s") * NC + lax.axis_index("c")
        idx_row0 = wid * idx_rows_per_w
        out_row0 = wid * rows_per_w

        def start_idx(cc, b):
            pltpu.async_copy(
                idx_hbm.at[pl.ds(idx_row0 + cc * GATHERS, GATHERS)],
                idx_v.at[b], sem_i.at[b])

        def wait_idx(cc, b):
            pltpu.make_async_copy(
                idx_hbm.at[pl.ds(idx_row0 + cc * GATHERS, GATHERS)],
                idx_v.at[b], sem_i.at[b]).wait()

        for b in range(NBUF):
            start_idx(b, b)

        def body(i, carry):
            for b in range(NBUF):
                cc = i * NBUF + b
                wait_idx(cc, b)

                @pl.when(cc >= NBUF)
                def _():
                    pltpu.make_async_copy(
                        rows_v.at[b],
                        out_hbm.at[pl.ds(out_row0 + (cc - NBUF) * CHUNK,
                                         CHUNK)],
                        sem_w.at[b]).wait()

                for g in range(GATHERS):
                    pltpu.async_copy(
                        table_hbm.at[idx_v.at[b].at[g]],
                        rows_v.at[b].at[pl.ds(g * IDX_W, IDX_W)], sem_g.at[b])
                for g in range(GATHERS):
                    pltpu.make_async_copy(
                        table_hbm.at[idx_v.at[b].at[g]],
                        rows_v.at[b].at[pl.ds(g * IDX_W, IDX_W)],
                        sem_g.at[b]).wait()
                start_idx(cc + NBUF, b)
                pltpu.async_copy(
                    rows_v.at[b],
                    out_hbm.at[pl.ds(out_row0 + cc * CHUNK, CHUNK)],
                    sem_w.at[b])
            return carry

        lax.fori_loop(0, chunks_per_w // NBUF, body, 0)

        for b in range(NBUF):
            cc = chunks_per_w - NBUF + b
            pltpu.make_async_copy(
                rows_v.at[b],
                out_hbm.at[pl.ds(out_row0 + cc * CHUNK, CHUNK)],
                sem_w.at[b]).wait()
            wait_idx(cc + NBUF, b)

    return k


def kernel(input_ids, table):
    BATCH, SEQ = input_ids.shape
    B = BATCH * SEQ
    info = plsc.get_sparse_core_info()
    NC, NS = info.num_cores, info.num_subcores
    idx2d = input_ids.reshape(B // IDX_W, IDX_W)
    pad = jnp.zeros((NBUF * GATHERS, IDX_W), jnp.int32)
    idx2d = jnp.concatenate([idx2d, pad], axis=0)
    k = _make_kernel(B, NC, NS)
    out = k(table, idx2d)
    return out.reshape(BATCH, SEQ, DIM)


# speedup vs baseline: 1.0455x; 1.0038x over previous
import jax
import jax.numpy as jnp
from jax import lax
from jax.experimental import pallas as pl
from jax.experimental.pallas import tpu as pltpu
from jax.experimental.pallas import tpu_sc as plsc

DIM = 64
IDX_W = 128
GATHERS = 4
CHUNK = IDX_W * GATHERS
NBUF = 2


def _make_kernel(B, NC, NS):
    NW = NC * NS
    rows_per_w = B // NW
    chunks_per_w = rows_per_w // CHUNK
    idx_rows_per_w = rows_per_w // IDX_W

    mesh = plsc.VectorSubcoreMesh(
        core_axis_name="c", subcore_axis_name="s",
        num_cores=NC, num_subcores=NS)

    @pl.kernel(
        out_type=jax.ShapeDtypeStruct((B, DIM), jnp.float32),
        mesh=mesh,
        scratch_types=[
            pltpu.VMEM((NBUF, GATHERS, IDX_W), jnp.int32),
            pltpu.VMEM((NBUF, CHUNK, DIM), jnp.float32),
            pltpu.SemaphoreType.DMA((NBUF,)),
            pltpu.SemaphoreType.DMA((NBUF,)),
            pltpu.SemaphoreType.DMA((NBUF,)),
        ],
        compiler_params=pltpu.CompilerParams(use_tc_tiling_on_sc=False),
    )
    def k(table_hbm, idx_hbm, out_hbm, idx_v, rows_v, sem_i, sem_g, sem_w):
        wid = lax.axis_index("s") * NC + lax.axis_index("c")
        idx_row0 = wid * idx_rows_per_w
        out_row0 = wid * rows_per_w

        def start_idx(cc, b):
            pltpu.async_copy(
                idx_hbm.at[pl.ds(idx_row0 + cc * GATHERS, GATHERS)],
                idx_v.at[b], sem_i.at[b])

        def wait_idx(cc, b):
            pltpu.make_async_copy(
                idx_hbm.at[pl.ds(idx_row0 + cc * GATHERS, GATHERS)],
                idx_v.at[b], sem_i.at[b]).wait()

        def fire_gathers(b):
            for g in range(GATHERS):
                pltpu.async_copy(
                    table_hbm.at[idx_v.at[b].at[g]],
                    rows_v.at[b].at[pl.ds(g * IDX_W, IDX_W)], sem_g.at[b])

        def wait_gathers(b):
            for g in range(GATHERS):
                pltpu.make_async_copy(
                    table_hbm.at[idx_v.at[b].at[g]],
                    rows_v.at[b].at[pl.ds(g * IDX_W, IDX_W)],
                    sem_g.at[b]).wait()

        def start_wb(cc, b):
            pltpu.async_copy(
                rows_v.at[b],
                out_hbm.at[pl.ds(out_row0 + cc * CHUNK, CHUNK)],
                sem_w.at[b])

        def wait_wb(cc, b):
            pltpu.make_async_copy(
                rows_v.at[b],
                out_hbm.at[pl.ds(out_row0 + cc * CHUNK, CHUNK)],
                sem_w.at[b]).wait()

        n = chunks_per_w

        start_idx(0, 0)
        start_idx(1, 1)
        wait_idx(0, 0)
        fire_gathers(0)

        def body(i, carry):
            for b in range(NBUF):
                c = i * NBUF + b
                bn = 1 - b
                wait_idx(c + 1, bn)

                @pl.when(c >= 1)
                def _():
                    wait_wb(c - 1, bn)

                @pl.when(c + 1 < n)
                def _():
                    fire_gathers(bn)

                wait_gathers(b)
                start_wb(c, b)
                start_idx(c + NBUF, b)
            return carry

        lax.fori_loop(0, n // NBUF, body, 0)

        wait_wb(n - 1, (n - 1) % NBUF)
        wait_idx(n + 1, (n + 1) % NBUF)

    return k


def kernel(input_ids, table):
    BATCH, SEQ = input_ids.shape
    B = BATCH * SEQ
    info = plsc.get_sparse_core_info()
    NC, NS = info.num_cores, info.num_subcores
    idx2d = input_ids.reshape(B // IDX_W, IDX_W)
    pad = jnp.zeros((NBUF * GATHERS, IDX_W), jnp.int32)
    idx2d = jnp.concatenate([idx2d, pad], axis=0)
    k = _make_kernel(B, NC, NS)
    out = k(table, idx2d)
    return out.reshape(BATCH, SEQ, DIM)
